# baseline (speedup 1.0000x reference)
import functools

import jax
import jax.numpy as jnp
from jax import lax
from jax.experimental import pallas as pl
from jax.experimental.pallas import tpu as pltpu
from jax.experimental.pallas import tpu_sc as plsc

_N = 50000
_E = 800000
_DC = 16
_EB = 2000
_NSC = 2
_NSUB = 16
_EPT = _E // _NSUB
_NBATCH = _EPT // _EB
_NP = 50048
_NROWS = _NP // _NSUB
_ZR = 782
_RB = 2000
_F32 = jnp.float32


@functools.lru_cache(maxsize=None)
def _make_segsum(num_chunks):
  mesh = plsc.VectorSubcoreMesh(
      core_axis_name="c", subcore_axis_name="s",
      num_cores=_NSC, num_subcores=_NSUB)
  out_type = jax.ShapeDtypeStruct((num_chunks, _NP, _DC), _F32)
  scratch_types = [
      pltpu.VMEM((2, _EB), jnp.int32),
      pltpu.VMEM((_EB, _DC), _F32),
      pltpu.VMEM((2, _EB), jnp.int32),
      pltpu.VMEM((_EB, _DC), _F32),
      pltpu.VMEM_SHARED((_NP, _DC), _F32),
      pltpu.SemaphoreType.DMA,
      pltpu.SemaphoreType.DMA,
      pltpu.SemaphoreType.DMA,
      pltpu.SemaphoreType.DMA,
      pltpu.SemaphoreType.DMA,
  ]

  def body(edg_hbm, zero_hbm, tab, out, idx_a, rows_a, idx_b, rows_b, acc,
           sem_g, sem_sa, sem_sb, sem_ia, sem_ib):
    core = lax.axis_index("c")
    sub = lax.axis_index("s")

    def chunk(cc):
      c = _NSC * cc + core
      for j in range(_NROWS // _ZR):
        pltpu.sync_copy(zero_hbm, acc.at[pl.ds(sub * _NROWS + j * _ZR, _ZR)])
      plsc.subcore_barrier()

      def slot(b, idx_v, rows_v, sem_s, sem_i):
        pltpu.make_async_copy(edg_hbm.at[b], idx_v, sem_i).wait()
        pltpu.async_copy(tab.at[c].at[idx_v.at[0]], rows_v, sem_g).wait()
        pltpu.async_copy(rows_v, acc.at[idx_v.at[1]], sem_s, add=True)

      def pair(k):
        b = sub * _NBATCH + 2 * k
        @pl.when(k > 0)
        def _():
          pltpu.make_async_copy(rows_a, acc.at[idx_a.at[1]], sem_sa).wait()
          pltpu.make_async_copy(rows_b, acc.at[idx_b.at[1]], sem_sb).wait()
        pltpu.async_copy(edg_hbm.at[b], idx_a, sem_ia)
        pltpu.async_copy(edg_hbm.at[b + 1], idx_b, sem_ib)
        slot(b, idx_a, rows_a, sem_sa, sem_ia)
        slot(b + 1, idx_b, rows_b, sem_sb, sem_ib)

      pl.loop(0, _NBATCH // 2)(pair)
      bl = sub * _NBATCH + _NBATCH - 1
      pltpu.make_async_copy(rows_a, acc.at[idx_a.at[1]], sem_sa).wait()
      pltpu.async_copy(edg_hbm.at[bl], idx_a, sem_ia)
      slot(bl, idx_a, rows_a, sem_sa, sem_ia)
      pltpu.make_async_copy(rows_a, acc.at[idx_a.at[1]], sem_sa).wait()
      pltpu.make_async_copy(rows_b, acc.at[idx_b.at[1]], sem_sb).wait()
      plsc.subcore_barrier()
      pltpu.sync_copy(acc.at[pl.ds(sub * _NROWS, _NROWS)],
                      out.at[c].at[pl.ds(sub * _NROWS, _NROWS)])

    nloc = (num_chunks + (_NSC - 1) - core) // _NSC
    pl.loop(0, nloc)(chunk)

  return pl.kernel(body, out_type=out_type, mesh=mesh,
                   scratch_types=scratch_types,
                   compiler_params=pltpu.CompilerParams(
                       use_tc_tiling_on_sc=False))


def _segsum(edg, zero_blk, table):
  w = table.shape[1]
  num_chunks = w // _DC
  tp = jnp.pad(table, ((0, _NP - _N), (0, 0)))
  t3d = tp.reshape(_NP, num_chunks, _DC).transpose(1, 0, 2)
  out = _make_segsum(num_chunks)(edg, zero_blk, t3d)
  return out.transpose(1, 0, 2).reshape(_NP, w)[:_N]


def _row_spec(width):
  return pl.BlockSpec((_RB, width), lambda i: (i, 0))


def _full_spec(r, c):
  return pl.BlockSpec((r, c), lambda i: (0, 0))


def _stats_update(h):
  s1 = jnp.sum(h, axis=0, keepdims=True)
  s2 = jnp.sum(h * h, axis=0, keepdims=True)
  pad = jnp.zeros((6, h.shape[1]), _F32)
  return jnp.concatenate([s1, s2, pad], axis=0)


def _tc_a1(v_ref, o_ref, w_ref, b_ref, h_ref, st_ref, y_ref):
  v = v_ref[...]
  o = o_ref[...]
  x = v + o
  h = jnp.dot(x, w_ref[...], preferred_element_type=_F32) + b_ref[...]
  h_ref[...] = h
  deg = o[:, 86:87] + 1.0
  dinv = lax.rsqrt(deg)
  y_ref[...] = v * dinv
  @pl.when(pl.program_id(0) == 0)
  def _():
    st_ref[...] = jnp.zeros_like(st_ref)
  st_ref[...] += _stats_update(h)


def _bn_relu(h, st, g, t):
  mean = st[0:1, :] * (1.0 / _N)
  var = st[1:2, :] * (1.0 / _N) - mean * mean
  scale = g * lax.rsqrt(var + 1e-5)
  shift = t - mean * scale
  return jnp.maximum(h * scale + shift, 0.0)


def _tc_bn_res(h_ref, st_ref, g_ref, t_ref, r_ref, x_ref):
  x_ref[...] = _bn_relu(h_ref[...], st_ref[...], g_ref[...],
                        t_ref[...]) + r_ref[...]


def _tc_gcn1(y_ref, s_ref, h1_ref, st_ref, g_ref, tt_ref, w_ref, b_ref,
             t3_ref):
  y = y_ref[...]
  dinv = y[:, 86:87]
  a = dinv * (y + s_ref[...])
  v1 = jnp.maximum(
      jnp.dot(a, w_ref[...], preferred_element_type=_F32) + b_ref[...], 0.0)
  x1 = _bn_relu(h1_ref[...], st_ref[...], g_ref[...], tt_ref[...])
  t3_ref[...] = jnp.concatenate([dinv * v1, x1 + v1], axis=1)


def _tc_gin_pre(t_ref, o_ref, w_ref, b_ref, h_ref, st_ref, *, half):
  xin = t_ref[:, half:] + o_ref[:, half:]
  h = jnp.dot(xin, w_ref[...], preferred_element_type=_F32) + b_ref[...]
  h_ref[...] = h
  @pl.when(pl.program_id(0) == 0)
  def _():
    st_ref[...] = jnp.zeros_like(st_ref)
  st_ref[...] += _stats_update(h)


def _tc_gcn_mid(t_ref, o_ref, h_ref, st_ref, g_ref, tt_ref, y_ref, w_ref,
                b_ref, t4_ref, *, half):
  dinv = y_ref[:, 86:87]
  a = dinv * (t_ref[:, :half] + o_ref[:, :half])
  vn = jnp.maximum(
      jnp.dot(a, w_ref[...], preferred_element_type=_F32) + b_ref[...], 0.0)
  x = _bn_relu(h_ref[...], st_ref[...], g_ref[...], tt_ref[...])
  t4_ref[...] = jnp.concatenate([dinv * vn, x + vn], axis=1)


def _tc_last(t_ref, o_ref, y_ref, wg_ref, bg_ref, we_ref, be_ref,
             h_ref, st_ref, v3_ref):
  xin = t_ref[:, 256:] + o_ref[:, 256:]
  h = jnp.dot(xin, wg_ref[...], preferred_element_type=_F32) + bg_ref[...]
  h_ref[...] = h
  dinv = y_ref[:, 86:87]
  a = dinv * (t_ref[:, :256] + o_ref[:, :256])
  v3_ref[...] = jnp.maximum(
      jnp.dot(a, we_ref[...], preferred_element_type=_F32) + be_ref[...], 0.0)
  @pl.when(pl.program_id(0) == 0)
  def _():
    st_ref[...] = jnp.zeros_like(st_ref)
  st_ref[...] += _stats_update(h)


def _grid_call(body, in_specs, out_shapes, out_specs):
  return pl.pallas_call(
      body,
      grid=(_N // _RB,),
      in_specs=in_specs,
      out_shape=out_shapes,
      out_specs=out_specs,
  )


def _pad_w(w):
  return jnp.pad(w, ((0, 96 - w.shape[0]), (0, 0)))


def kernel(v, edges, Wg1, bg1, gm1, bt1, Wg2, bg2, gm2, bt2, Wg3, bg3, gm3,
           bt3, We1, be1, We2, be2, We3, be3):
  n = v.shape[0]
  edg = edges.reshape(2, _E // _EB, _EB).transpose(1, 0, 2)
  zero_blk = jnp.zeros((_ZR, _DC), _F32)

  ones = jnp.ones((n, 1), _F32)
  zpad = jnp.zeros((n, 9), _F32)
  v_pad = jnp.concatenate([v, ones, zpad], axis=1)

  wg1p, we1p = _pad_w(Wg1), _pad_w(We1)
  bg1r, be1r = bg1.reshape(1, -1), be1.reshape(1, -1)
  bg2r, be2r = bg2.reshape(1, -1), be2.reshape(1, -1)
  bg3r, be3r = bg3.reshape(1, -1), be3.reshape(1, -1)
  gm1r, bt1r = gm1.reshape(1, -1), bt1.reshape(1, -1)
  gm2r, bt2r = gm2.reshape(1, -1), bt2.reshape(1, -1)
  gm3r, bt3r = gm3.reshape(1, -1), bt3.reshape(1, -1)

  o1 = _segsum(edg, zero_blk, v_pad)

  h1, st1, y0p = _grid_call(
      _tc_a1,
      [_row_spec(96), _row_spec(96), _full_spec(96, 128), _full_spec(1, 128)],
      (jax.ShapeDtypeStruct((n, 128), _F32),
       jax.ShapeDtypeStruct((8, 128), _F32),
       jax.ShapeDtypeStruct((n, 96), _F32)),
      (_row_spec(128), _full_spec(8, 128), _row_spec(96)),
  )(v_pad, o1, wg1p, bg1r)

  s0 = _segsum(edg, zero_blk, y0p)

  t3 = _grid_call(
      _tc_gcn1,
      [_row_spec(96), _row_spec(96), _row_spec(128), _full_spec(8, 128),
       _full_spec(1, 128), _full_spec(1, 128), _full_spec(96, 128),
       _full_spec(1, 128)],
      jax.ShapeDtypeStruct((n, 256), _F32),
      _row_spec(256),
  )(y0p, s0, h1, st1, gm1r, bt1r, we1p, be1r)

  o3 = _segsum(edg, zero_blk, t3)

  h2, st2 = _grid_call(
      functools.partial(_tc_gin_pre, half=128),
      [_row_spec(256), _row_spec(256), _full_spec(128, 256),
       _full_spec(1, 256)],
      (jax.ShapeDtypeStruct((n, 256), _F32),
       jax.ShapeDtypeStruct((8, 256), _F32)),
      (_row_spec(256), _full_spec(8, 256)),
  )(t3, o3, Wg2, bg2r)

  t4 = _grid_call(
      functools.partial(_tc_gcn_mid, half=128),
      [_row_spec(256), _row_spec(256), _row_spec(256), _full_spec(8, 256),
       _full_spec(1, 256), _full_spec(1, 256), _row_spec(96),
       _full_spec(128, 256), _full_spec(1, 256)],
      jax.ShapeDtypeStruct((n, 512), _F32),
      _row_spec(512),
  )(t3, o3, h2, st2, gm2r, bt2r, y0p, We2, be2r)

  o4 = _segsum(edg, zero_blk, t4)

  h3, st3, v3 = _grid_call(
      _tc_last,
      [_row_spec(512), _row_spec(512), _row_spec(96), _full_spec(256, 512),
       _full_spec(1, 512), _full_spec(256, 512), _full_spec(1, 512)],
      (jax.ShapeDtypeStruct((n, 512), _F32),
       jax.ShapeDtypeStruct((8, 512), _F32),
       jax.ShapeDtypeStruct((n, 512), _F32)),
      (_row_spec(512), _full_spec(8, 512), _row_spec(512)),
  )(t4, o4, y0p, Wg3, bg3r, We3, be3r)

  out = _grid_call(
      _tc_bn_res,
      [_row_spec(512), _full_spec(8, 512), _full_spec(1, 512),
       _full_spec(1, 512), _row_spec(512)],
      jax.ShapeDtypeStruct((n, 512), _F32),
      _row_spec(512),
  )(h3, st3, gm3r, bt3r, v3)

  return out


# speedup vs baseline: 1.0223x; 1.0223x over previous
import functools

import jax
import jax.numpy as jnp
from jax import lax
from jax.experimental import pallas as pl
from jax.experimental.pallas import tpu as pltpu
from jax.experimental.pallas import tpu_sc as plsc

_N = 50000
_E = 800000
_DC = 16
_EB = 2000
_NSC = 2
_NSUB = 16
_EPT = _E // _NSUB
_NBATCH = _EPT // _EB
_NP = 50048
_NROWS = _NP // _NSUB
_ZR = 782
_RB = 2000
_F32 = jnp.float32


@functools.lru_cache(maxsize=None)
def _make_segsum(num_chunks):
  mesh = plsc.VectorSubcoreMesh(
      core_axis_name="c", subcore_axis_name="s",
      num_cores=_NSC, num_subcores=_NSUB)
  out_type = jax.ShapeDtypeStruct((num_chunks, _NP, _DC), _F32)
  scratch_types = [
      pltpu.VMEM((2, _EB), jnp.int32),
      pltpu.VMEM((_EB, _DC), _F32),
      pltpu.VMEM((2, _EB), jnp.int32),
      pltpu.VMEM((_EB, _DC), _F32),
      pltpu.VMEM_SHARED((_NP, _DC), _F32),
      pltpu.SemaphoreType.DMA,
      pltpu.SemaphoreType.DMA,
      pltpu.SemaphoreType.DMA,
  ]

  def body(edg_hbm, zero_hbm, tab, out, idx_a, rows_a, idx_b, rows_b, acc,
           sem_g, sem_sa, sem_sb):
    core = lax.axis_index("c")
    sub = lax.axis_index("s")

    def chunk(cc):
      c = _NSC * cc + core
      for j in range(_NROWS // _ZR):
        pltpu.sync_copy(zero_hbm, acc.at[pl.ds(sub * _NROWS + j * _ZR, _ZR)])
      plsc.subcore_barrier()

      def slot(k, b, idx_v, rows_v, sem_s):
        @pl.when(k > 0)
        def _():
          pltpu.make_async_copy(rows_v, acc.at[idx_v.at[1]], sem_s).wait()
        pltpu.sync_copy(edg_hbm.at[b], idx_v)
        pltpu.async_copy(tab.at[c].at[idx_v.at[0]], rows_v, sem_g).wait()
        pltpu.async_copy(rows_v, acc.at[idx_v.at[1]], sem_s, add=True)

      def pair(k):
        b = sub * _NBATCH + 2 * k
        slot(k, b, idx_a, rows_a, sem_sa)
        slot(k, b + 1, idx_b, rows_b, sem_sb)

      pl.loop(0, _NBATCH // 2)(pair)
      slot(jnp.int32(_NBATCH // 2), sub * _NBATCH + _NBATCH - 1,
           idx_a, rows_a, sem_sa)
      pltpu.make_async_copy(rows_a, acc.at[idx_a.at[1]], sem_sa).wait()
      pltpu.make_async_copy(rows_b, acc.at[idx_b.at[1]], sem_sb).wait()
      plsc.subcore_barrier()
      pltpu.sync_copy(acc.at[pl.ds(sub * _NROWS, _NROWS)],
                      out.at[c].at[pl.ds(sub * _NROWS, _NROWS)])

    nloc = (num_chunks + (_NSC - 1) - core) // _NSC
    pl.loop(0, nloc)(chunk)

  return pl.kernel(body, out_type=out_type, mesh=mesh,
                   scratch_types=scratch_types,
                   compiler_params=pltpu.CompilerParams(
                       use_tc_tiling_on_sc=False))


def _segsum(edg, zero_blk, table):
  w = table.shape[1]
  num_chunks = w // _DC
  tp = jnp.pad(table, ((0, _NP - _N), (0, 0)))
  t3d = tp.reshape(_NP, num_chunks, _DC).transpose(1, 0, 2)
  out = _make_segsum(num_chunks)(edg, zero_blk, t3d)
  return out.transpose(1, 0, 2).reshape(_NP, w)[:_N]


def _row_spec(width):
  return pl.BlockSpec((_RB, width), lambda i: (i, 0))


def _full_spec(r, c):
  return pl.BlockSpec((r, c), lambda i: (0, 0))


def _stats_update(h):
  s1 = jnp.sum(h, axis=0, keepdims=True)
  s2 = jnp.sum(h * h, axis=0, keepdims=True)
  pad = jnp.zeros((6, h.shape[1]), _F32)
  return jnp.concatenate([s1, s2, pad], axis=0)


def _tc_a1(v_ref, o_ref, w_ref, b_ref, h_ref, st_ref, y_ref):
  v = v_ref[...]
  o = o_ref[...]
  x = v + o
  h = jnp.dot(x, w_ref[...], preferred_element_type=_F32) + b_ref[...]
  h_ref[...] = h
  deg = o[:, 86:87] + 1.0
  dinv = lax.rsqrt(deg)
  y_ref[...] = v * dinv
  @pl.when(pl.program_id(0) == 0)
  def _():
    st_ref[...] = jnp.zeros_like(st_ref)
  st_ref[...] += _stats_update(h)


def _bn_relu(h, st, g, t):
  mean = st[0:1, :] * (1.0 / _N)
  var = st[1:2, :] * (1.0 / _N) - mean * mean
  scale = g * lax.rsqrt(var + 1e-5)
  shift = t - mean * scale
  return jnp.maximum(h * scale + shift, 0.0)


def _tc_bn_res(h_ref, st_ref, g_ref, t_ref, r_ref, x_ref):
  x_ref[...] = _bn_relu(h_ref[...], st_ref[...], g_ref[...],
                        t_ref[...]) + r_ref[...]


def _tc_gcn1(y_ref, s_ref, h1_ref, st_ref, g_ref, tt_ref, w_ref, b_ref,
             t3_ref):
  y = y_ref[...]
  dinv = y[:, 86:87]
  a = dinv * (y + s_ref[...])
  v1 = jnp.maximum(
      jnp.dot(a, w_ref[...], preferred_element_type=_F32) + b_ref[...], 0.0)
  x1 = _bn_relu(h1_ref[...], st_ref[...], g_ref[...], tt_ref[...])
  t3_ref[...] = jnp.concatenate([dinv * v1, x1 + v1], axis=1)


def _tc_gin_pre(t_ref, o_ref, w_ref, b_ref, h_ref, st_ref, *, half):
  xin = t_ref[:, half:] + o_ref[:, half:]
  h = jnp.dot(xin, w_ref[...], preferred_element_type=_F32) + b_ref[...]
  h_ref[...] = h
  @pl.when(pl.program_id(0) == 0)
  def _():
    st_ref[...] = jnp.zeros_like(st_ref)
  st_ref[...] += _stats_update(h)


def _tc_gcn_mid(t_ref, o_ref, h_ref, st_ref, g_ref, tt_ref, y_ref, w_ref,
                b_ref, t4_ref, *, half):
  dinv = y_ref[:, 86:87]
  a = dinv * (t_ref[:, :half] + o_ref[:, :half])
  vn = jnp.maximum(
      jnp.dot(a, w_ref[...], preferred_element_type=_F32) + b_ref[...], 0.0)
  x = _bn_relu(h_ref[...], st_ref[...], g_ref[...], tt_ref[...])
  t4_ref[...] = jnp.concatenate([dinv * vn, x + vn], axis=1)


def _tc_last(t_ref, o_ref, y_ref, wg_ref, bg_ref, we_ref, be_ref,
             h_ref, st_ref, v3_ref):
  xin = t_ref[:, 256:] + o_ref[:, 256:]
  h = jnp.dot(xin, wg_ref[...], preferred_element_type=_F32) + bg_ref[...]
  h_ref[...] = h
  dinv = y_ref[:, 86:87]
  a = dinv * (t_ref[:, :256] + o_ref[:, :256])
  v3_ref[...] = jnp.maximum(
      jnp.dot(a, we_ref[...], preferred_element_type=_F32) + be_ref[...], 0.0)
  @pl.when(pl.program_id(0) == 0)
  def _():
    st_ref[...] = jnp.zeros_like(st_ref)
  st_ref[...] += _stats_update(h)


def _grid_call(body, in_specs, out_shapes, out_specs):
  return pl.pallas_call(
      body,
      grid=(_N // _RB,),
      in_specs=in_specs,
      out_shape=out_shapes,
      out_specs=out_specs,
  )


def _pad_w(w):
  return jnp.pad(w, ((0, 96 - w.shape[0]), (0, 0)))


def kernel(v, edges, Wg1, bg1, gm1, bt1, Wg2, bg2, gm2, bt2, Wg3, bg3, gm3,
           bt3, We1, be1, We2, be2, We3, be3):
  n = v.shape[0]
  edg = edges.reshape(2, _E // _EB, _EB).transpose(1, 0, 2)
  zero_blk = jnp.zeros((_ZR, _DC), _F32)

  ones = jnp.ones((n, 1), _F32)
  zpad = jnp.zeros((n, 9), _F32)
  v_pad = jnp.concatenate([v, ones, zpad], axis=1)

  wg1p, we1p = _pad_w(Wg1), _pad_w(We1)
  bg1r, be1r = bg1.reshape(1, -1), be1.reshape(1, -1)
  bg2r, be2r = bg2.reshape(1, -1), be2.reshape(1, -1)
  bg3r, be3r = bg3.reshape(1, -1), be3.reshape(1, -1)
  gm1r, bt1r = gm1.reshape(1, -1), bt1.reshape(1, -1)
  gm2r, bt2r = gm2.reshape(1, -1), bt2.reshape(1, -1)
  gm3r, bt3r = gm3.reshape(1, -1), bt3.reshape(1, -1)

  o1 = _segsum(edg, zero_blk, v_pad)

  h1, st1, y0p = _grid_call(
      _tc_a1,
      [_row_spec(96), _row_spec(96), _full_spec(96, 128), _full_spec(1, 128)],
      (jax.ShapeDtypeStruct((n, 128), _F32),
       jax.ShapeDtypeStruct((8, 128), _F32),
       jax.ShapeDtypeStruct((n, 96), _F32)),
      (_row_spec(128), _full_spec(8, 128), _row_spec(96)),
  )(v_pad, o1, wg1p, bg1r)

  s0 = _segsum(edg, zero_blk, y0p)

  t3 = _grid_call(
      _tc_gcn1,
      [_row_spec(96), _row_spec(96), _row_spec(128), _full_spec(8, 128),
       _full_spec(1, 128), _full_spec(1, 128), _full_spec(96, 128),
       _full_spec(1, 128)],
      jax.ShapeDtypeStruct((n, 256), _F32),
      _row_spec(256),
  )(y0p, s0, h1, st1, gm1r, bt1r, we1p, be1r)

  o3 = _segsum(edg, zero_blk, t3)

  h2, st2 = _grid_call(
      functools.partial(_tc_gin_pre, half=128),
      [_row_spec(256), _row_spec(256), _full_spec(128, 256),
       _full_spec(1, 256)],
      (jax.ShapeDtypeStruct((n, 256), _F32),
       jax.ShapeDtypeStruct((8, 256), _F32)),
      (_row_spec(256), _full_spec(8, 256)),
  )(t3, o3, Wg2, bg2r)

  t4 = _grid_call(
      functools.partial(_tc_gcn_mid, half=128),
      [_row_spec(256), _row_spec(256), _row_spec(256), _full_spec(8, 256),
       _full_spec(1, 256), _full_spec(1, 256), _row_spec(96),
       _full_spec(128, 256), _full_spec(1, 256)],
      jax.ShapeDtypeStruct((n, 512), _F32),
      _row_spec(512),
  )(t3, o3, h2, st2, gm2r, bt2r, y0p, We2, be2r)

  o4 = _segsum(edg, zero_blk, t4)

  h3, st3, v3 = _grid_call(
      _tc_last,
      [_row_spec(512), _row_spec(512), _row_spec(96), _full_spec(256, 512),
       _full_spec(1, 512), _full_spec(256, 512), _full_spec(1, 512)],
      (jax.ShapeDtypeStruct((n, 512), _F32),
       jax.ShapeDtypeStruct((8, 512), _F32),
       jax.ShapeDtypeStruct((n, 512), _F32)),
      (_row_spec(512), _full_spec(8, 512), _row_spec(512)),
  )(t4, o4, y0p, Wg3, bg3r, We3, be3r)

  out = _grid_call(
      _tc_bn_res,
      [_row_spec(512), _full_spec(8, 512), _full_spec(1, 512),
       _full_spec(1, 512), _row_spec(512)],
      jax.ShapeDtypeStruct((n, 512), _F32),
      _row_spec(512),
  )(h3, st3, gm3r, bt3r, v3)

  return out
